# baseline (speedup 1.0000x reference)
import functools

import jax
import jax.numpy as jnp
from jax import lax
from jax.experimental import pallas as pl
from jax.experimental.pallas import tpu as pltpu
from jax.experimental.pallas import tpu_sc as plsc

NC = 2
NS = 16
NW = NC * NS

BW = 128
NBUF = 4


def _body(idx_hbm, table_hbm, out_hbm, idx_v, idx2_v, rows_v, ost_v,
          gsem, ssem, *, seq, embed):
    c = lax.axis_index("c")
    s = lax.axis_index("s")
    wid = c * NS + s
    bcol = wid * BW

    pltpu.sync_copy(idx_hbm.at[:, pl.ds(bcol, BW)], idx_v)

    ridx = [lax.iota(jnp.int32, 16) + 16 * k for k in range(8)]

    def gather(j):
        return pltpu.make_async_copy(
            table_hbm.at[idx2_v.at[j]], rows_v.at[j], gsem.at[j])

    def scatter(g, j):
        return pltpu.make_async_copy(
            ost_v.at[j], out_hbm.at[g, :, pl.ds(bcol, BW)], ssem.at[j])

    def prep_gather(g, j):
        for k in range(8):
            v = idx_v[g, pl.ds(16 * k, 16)]
            idx2_v[j, pl.ds(16 * k, 16)] = v >> 1
        gather(j).start()

    def produce(g, j):
        pbase = []
        for k in range(8):
            v = idx_v[g, pl.ds(16 * k, 16)]
            pbase.append((v & 1) << 6)

        @plsc.parallel_loop(0, embed, step=1, unroll=16)
        def _(e):
            for k in range(8):
                col = pbase[k] + e
                val = plsc.load_gather(rows_v.at[j], [ridx[k], col])
                ost_v[j, e, pl.ds(16 * k, 16)] = val

        scatter(g, j).start()

    for j in range(NBUF):
        prep_gather(j, j)

    for j in range(NBUF):
        gather(j).wait()
        produce(j, j)
        prep_gather(j + NBUF, j)

    def pass_body(k, carry):
        g0 = k * NBUF
        for j in range(NBUF):
            gather(j).wait()
            scatter(g0 + j, j).wait()
            produce(g0 + j, j)
            prep_gather(g0 + j + NBUF, j)
        return carry

    lax.fori_loop(1, seq // NBUF - 1, pass_body, 0)

    g0 = seq - NBUF
    for j in range(NBUF):
        gather(j).wait()
        scatter(g0 + j, j).wait()
        produce(g0 + j, j)
    for j in range(NBUF):
        scatter(g0 + j, j).wait()


def kernel(word_batch, table):
    b, sq = word_batch.shape
    v, d = table.shape

    idx_t = word_batch.T
    table2 = table.reshape(v // 2, 2 * d)

    mesh = plsc.VectorSubcoreMesh(core_axis_name="c", subcore_axis_name="s")
    body = functools.partial(_body, seq=sq, embed=d)
    out_t = pl.kernel(
        body,
        out_type=jax.ShapeDtypeStruct((sq, d, b), jnp.float32),
        mesh=mesh,
        scratch_types=[
            pltpu.VMEM((sq, BW), jnp.int32),
            pltpu.VMEM((NBUF, BW), jnp.int32),
            pltpu.VMEM((NBUF, BW, 2 * d), jnp.float32),
            pltpu.VMEM((NBUF, d, BW), jnp.float32),
            pltpu.SemaphoreType.DMA((NBUF,)),
            pltpu.SemaphoreType.DMA((NBUF,)),
        ],
        compiler_params=pltpu.CompilerParams(needs_layout_passes=False),
    )(idx_t, table2)
    return out_t.transpose(2, 0, 1)


# speedup vs baseline: 1.0019x; 1.0019x over previous
import functools

import jax
import jax.numpy as jnp
from jax import lax
from jax.experimental import pallas as pl
from jax.experimental.pallas import tpu as pltpu
from jax.experimental.pallas import tpu_sc as plsc

NC = 2
NS = 16
NW = NC * NS

BW = 128
NBUF = 4


def _body(idx_hbm, table_hbm, out_hbm, idx_v, idx2_v, rows_v, ost_v,
          gsem, ssem, *, seq, embed):
    c = lax.axis_index("c")
    s = lax.axis_index("s")
    wid = c * NS + s
    bcol = wid * BW

    pltpu.sync_copy(idx_hbm.at[:, pl.ds(bcol, BW)], idx_v)

    ridx = [lax.iota(jnp.int32, 16) + 16 * k for k in range(8)]

    def gather(j):
        return pltpu.make_async_copy(
            table_hbm.at[idx2_v.at[j]], rows_v.at[j], gsem.at[j])

    def scatter(g, j):
        return pltpu.make_async_copy(
            ost_v.at[j], out_hbm.at[g, :, pl.ds(bcol, BW)], ssem.at[j])

    def prep_gather(g, j):
        for k in range(8):
            v = idx_v[g, pl.ds(16 * k, 16)]
            idx2_v[j, pl.ds(16 * k, 16)] = v >> 1
        gather(j).start()

    def produce(g, j):
        pbase = []
        for k in range(8):
            v = idx_v[g, pl.ds(16 * k, 16)]
            pbase.append((v & 1) << 6)

        @plsc.parallel_loop(0, embed, step=1, unroll=2)
        def _(e):
            for k in range(8):
                col = pbase[k] + e
                val = plsc.load_gather(rows_v.at[j], [ridx[k], col])
                ost_v[j, e, pl.ds(16 * k, 16)] = val

        scatter(g, j).start()

    for j in range(NBUF):
        prep_gather(j, j)

    for j in range(NBUF):
        gather(j).wait()
        produce(j, j)
        prep_gather(j + NBUF, j)

    def pass_body(k, carry):
        g0 = k * NBUF
        for j in range(NBUF):
            gather(j).wait()
            scatter(g0 + j, j).wait()
            produce(g0 + j, j)
            prep_gather(g0 + j + NBUF, j)
        return carry

    lax.fori_loop(1, seq // NBUF - 1, pass_body, 0)

    g0 = seq - NBUF
    for j in range(NBUF):
        gather(j).wait()
        scatter(g0 + j, j).wait()
        produce(g0 + j, j)
    for j in range(NBUF):
        scatter(g0 + j, j).wait()


def kernel(word_batch, table):
    b, sq = word_batch.shape
    v, d = table.shape

    idx_t = word_batch.T
    table2 = table.reshape(v // 2, 2 * d)

    mesh = plsc.VectorSubcoreMesh(core_axis_name="c", subcore_axis_name="s")
    body = functools.partial(_body, seq=sq, embed=d)
    out_t = pl.kernel(
        body,
        out_type=jax.ShapeDtypeStruct((sq, d, b), jnp.float32),
        mesh=mesh,
        scratch_types=[
            pltpu.VMEM((sq, BW), jnp.int32),
            pltpu.VMEM((NBUF, BW), jnp.int32),
            pltpu.VMEM((NBUF, BW, 2 * d), jnp.float32),
            pltpu.VMEM((NBUF, d, BW), jnp.float32),
            pltpu.SemaphoreType.DMA((NBUF,)),
            pltpu.SemaphoreType.DMA((NBUF,)),
        ],
        compiler_params=pltpu.CompilerParams(needs_layout_passes=False),
    )(idx_t, table2)
    return out_t.transpose(2, 0, 1)


# speedup vs baseline: 1.0237x; 1.0218x over previous
import functools

import jax
import jax.numpy as jnp
from jax import lax
from jax.experimental import pallas as pl
from jax.experimental.pallas import tpu as pltpu
from jax.experimental.pallas import tpu_sc as plsc

NC = 2
NS = 16
NW = NC * NS

GROUP = 128
NBUF = 8


def _body(idx_hbm, table_hbm, out_hbm, idx_v, rows_v, gsem, ssem,
          *, gpw, rpw, steps):
    c = lax.axis_index("c")
    s = lax.axis_index("s")
    wid = c * NS + s
    gbase = wid * gpw
    rbase = wid * rpw

    pltpu.sync_copy(idx_hbm.at[pl.ds(gbase, gpw)], idx_v)

    def gather(g, j):
        return pltpu.make_async_copy(
            table_hbm.at[idx_v.at[g]], rows_v.at[j], gsem.at[j])

    def scatter(g, j):
        off = pl.multiple_of(rbase + g * GROUP, GROUP)
        return pltpu.make_async_copy(
            rows_v.at[j], out_hbm.at[pl.ds(off, GROUP)], ssem.at[j])

    for j in range(NBUF):
        gather(j, j).start()

    def loop_body(k, carry):
        g0 = k * NBUF
        for j in range(NBUF):
            g = g0 + j
            gather(g, j).wait()
            scatter(g, j).start()
        for j in range(NBUF):
            gn = g0 + NBUF + j
            scatter(g0 + j, j).wait()
            gather(gn, j).start()
        return carry

    lax.fori_loop(0, steps // NBUF - 1, loop_body, 0)

    g0 = steps - NBUF
    for j in range(NBUF):
        g = g0 + j
        gather(g, j).wait()
        scatter(g, j).start()
    for j in range(NBUF):
        scatter(g0 + j, j).wait()


def kernel(word_batch, table):
    b, sq = word_batch.shape
    n = b * sq
    d = table.shape[1]
    rpw = n // NW
    steps = rpw // GROUP
    gpw = steps

    idx = word_batch.reshape(n // GROUP, GROUP).astype(jnp.int32)

    mesh = plsc.VectorSubcoreMesh(core_axis_name="c", subcore_axis_name="s")
    body = functools.partial(_body, gpw=gpw, rpw=rpw, steps=steps)
    out = pl.kernel(
        body,
        out_type=jax.ShapeDtypeStruct((n, d), jnp.float32),
        mesh=mesh,
        scratch_types=[
            pltpu.VMEM((gpw, GROUP), jnp.int32),
            pltpu.VMEM((NBUF, GROUP, d), jnp.float32),
            pltpu.SemaphoreType.DMA((NBUF,)),
            pltpu.SemaphoreType.DMA((NBUF,)),
        ],
        compiler_params=pltpu.CompilerParams(use_tc_tiling_on_sc=False),
    )(idx, table)
    return out.reshape(b, sq, d)
